# initial kernel scaffold (unmeasured)
import jax
import jax.numpy as jnp
from jax import lax
from jax.experimental import pallas as pl
from jax.experimental.pallas import tpu as pltpu

N_DEV = 16
B = 2
SQ_SHARD = 128
SQ = N_DEV * SQ_SHARD
D = 512
H_LOC = 8
DH = 64
SKV = 128
SCALE = 0.125


def kernel(x, Wq, Wo, K_ext, V_ext):
    my = lax.axis_index("i")
    k_loc = lax.dynamic_slice(
        K_ext.reshape(B, SKV, 128 * DH), (0, 0, my * (H_LOC * DH)),
        (B, SKV, H_LOC * DH))
    v_loc = lax.dynamic_slice(
        V_ext.reshape(B, SKV, 128 * DH), (0, 0, my * (H_LOC * DH)),
        (B, SKV, H_LOC * DH))

    def body(x_ref, wq_ref, wo_ref, k_ref, v_ref, out_ref,
             xf, qb, ao, pt, agc, rsc,
             ag_send, ag_recv, rs_send, rs_recv):
        my = lax.axis_index("i")
        left = (my - 1) % N_DEV
        right = (my + 1) % N_DEV

        barrier_sem = pltpu.get_barrier_semaphore()
        for nbr in [left, right]:
            pl.semaphore_signal(
                barrier_sem, inc=1,
                device_id=(nbr,), device_id_type=pl.DeviceIdType.MESH)
        pl.semaphore_wait(barrier_sem, 2)

        xf[:, pl.ds(my * SQ_SHARD, SQ_SHARD), :] = x_ref[...]
        agc[0] = x_ref[...]
        for h in range(N_DEV - 1):
            s_slot = h % 2
            r_slot = (h + 1) % 2
            rdma = pltpu.make_async_remote_copy(
                src_ref=agc.at[s_slot],
                dst_ref=agc.at[r_slot],
                send_sem=ag_send.at[s_slot],
                recv_sem=ag_recv.at[r_slot],
                device_id=(right,),
                device_id_type=pl.DeviceIdType.MESH,
            )
            rdma.start()
            rdma.wait()
            origin = (my - h - 1) % N_DEV
            xf[:, pl.ds(origin * SQ_SHARD, SQ_SHARD), :] = agc[r_slot]

        wq = wq_ref[...]
        wo = wo_ref[...]
        for b in range(B):
            qb[b, :, :] = jnp.dot(xf[b, :, :], wq,
                                  preferred_element_type=jnp.float32)
        for b in range(B):
            for h in range(H_LOC):
                c0 = h * DH
                q = qb[b, :, c0:c0 + DH]
                k = k_ref[b, :, c0:c0 + DH]
                v = v_ref[b, :, c0:c0 + DH]
                s = lax.dot_general(
                    q, k, (((1,), (1,)), ((), ())),
                    preferred_element_type=jnp.float32) * SCALE
                m = jnp.max(s, axis=-1, keepdims=True)
                p = jnp.exp(s - m)
                l = jnp.sum(p, axis=-1, keepdims=True)
                o = jnp.dot(p, v, preferred_element_type=jnp.float32) / l
                ao[b, :, c0:c0 + DH] = o
        for b in range(B):
            pt[b, :, :] = jnp.dot(ao[b, :, :], wo,
                                  preferred_element_type=jnp.float32)

        for st in range(N_DEV - 1):
            s_slot = st % 2
            r_slot = (st + 1) % 2
            j = (my - st - 1) % N_DEV
            blk = pt[:, pl.ds(j * SQ_SHARD, SQ_SHARD), :]
            if st == 0:
                rsc[s_slot] = blk
            else:
                rsc[s_slot] = rsc[s_slot] + blk
            rdma = pltpu.make_async_remote_copy(
                src_ref=rsc.at[s_slot],
                dst_ref=rsc.at[r_slot],
                send_sem=rs_send.at[s_slot],
                recv_sem=rs_recv.at[r_slot],
                device_id=(right,),
                device_id_type=pl.DeviceIdType.MESH,
            )
            rdma.start()
            rdma.wait()
        out_ref[...] = pt[:, pl.ds(my * SQ_SHARD, SQ_SHARD), :] + rsc[1]

    return pl.pallas_call(
        body,
        out_shape=jax.ShapeDtypeStruct((B, SQ_SHARD, D), jnp.float32),
        in_specs=[pl.BlockSpec(memory_space=pltpu.VMEM)] * 5,
        out_specs=pl.BlockSpec(memory_space=pltpu.VMEM),
        scratch_shapes=[
            pltpu.VMEM((B, SQ, D), jnp.float32),
            pltpu.VMEM((B, SQ, D), jnp.float32),
            pltpu.VMEM((B, SQ, D), jnp.float32),
            pltpu.VMEM((B, SQ, D), jnp.float32),
            pltpu.VMEM((2, B, SQ_SHARD, D), jnp.float32),
            pltpu.VMEM((2, B, SQ_SHARD, D), jnp.float32),
            pltpu.SemaphoreType.DMA((2,)),
            pltpu.SemaphoreType.DMA((2,)),
            pltpu.SemaphoreType.DMA((2,)),
            pltpu.SemaphoreType.DMA((2,)),
        ],
        compiler_params=pltpu.CompilerParams(collective_id=0),
    )(x, Wq, Wo, k_loc, v_loc)


# baseline (device time: 276288 ns/iter reference)
import jax
import jax.numpy as jnp
from jax import lax
from jax.experimental import pallas as pl
from jax.experimental.pallas import tpu as pltpu

N_DEV = 16
B = 2
SQ_SHARD = 128
SQ = N_DEV * SQ_SHARD
D = 512
H_LOC = 8
DH = 64
SKV = 128
SCALE = 0.125


def kernel(x, Wq, Wo, K_ext, V_ext):
    my = lax.axis_index("i")
    k_loc = lax.dynamic_slice(
        K_ext.reshape(B, SKV, 128 * DH), (0, 0, my * (H_LOC * DH)),
        (B, SKV, H_LOC * DH))
    v_loc = lax.dynamic_slice(
        V_ext.reshape(B, SKV, 128 * DH), (0, 0, my * (H_LOC * DH)),
        (B, SKV, H_LOC * DH))

    def body(x_ref, wq_ref, wo_ref, k_ref, v_ref, out_ref,
             xf, pt, agc, rsc,
             ag_send, ag_recv, rs_send, rs_recv):
        my = lax.axis_index("i")
        left = (my - 1) % N_DEV
        right = (my + 1) % N_DEV

        barrier_sem = pltpu.get_barrier_semaphore()
        for nbr in [left, right]:
            pl.semaphore_signal(
                barrier_sem, inc=1,
                device_id=(nbr,), device_id_type=pl.DeviceIdType.MESH)
        pl.semaphore_wait(barrier_sem, 2)

        xf[:, pl.ds(my * SQ_SHARD, SQ_SHARD), :] = x_ref[...]
        agc[0] = x_ref[...]
        for h in range(N_DEV - 1):
            s_slot = h % 2
            r_slot = (h + 1) % 2
            rdma = pltpu.make_async_remote_copy(
                src_ref=agc.at[s_slot],
                dst_ref=agc.at[r_slot],
                send_sem=ag_send.at[s_slot],
                recv_sem=ag_recv.at[r_slot],
                device_id=(right,),
                device_id_type=pl.DeviceIdType.MESH,
            )
            rdma.start()
            rdma.wait()
            origin = (my - h - 1) % N_DEV
            xf[:, pl.ds(origin * SQ_SHARD, SQ_SHARD), :] = agc[r_slot]

        for b in range(B):
            for h in range(H_LOC):
                c0 = h * DH
                q = jnp.dot(xf[b, :, :], wq_ref[:, c0:c0 + DH],
                            preferred_element_type=jnp.float32)
                k = k_ref[b, :, c0:c0 + DH]
                v = v_ref[b, :, c0:c0 + DH]
                s = lax.dot_general(
                    q, k, (((1,), (1,)), ((), ())),
                    preferred_element_type=jnp.float32) * SCALE
                m = jnp.max(s, axis=-1, keepdims=True)
                p = jnp.exp(s - m)
                l = jnp.sum(p, axis=-1, keepdims=True)
                o = jnp.dot(p, v, preferred_element_type=jnp.float32) / l
                po = jnp.dot(o, wo_ref[c0:c0 + DH, :],
                             preferred_element_type=jnp.float32)
                if h == 0:
                    pt[b, :, :] = po
                else:
                    pt[b, :, :] = pt[b, :, :] + po

        for st in range(N_DEV - 1):
            s_slot = st % 2
            r_slot = (st + 1) % 2
            j = (my - st - 1) % N_DEV
            blk = pt[:, pl.ds(j * SQ_SHARD, SQ_SHARD), :]
            if st == 0:
                rsc[s_slot] = blk
            else:
                rsc[s_slot] = rsc[s_slot] + blk
            rdma = pltpu.make_async_remote_copy(
                src_ref=rsc.at[s_slot],
                dst_ref=rsc.at[r_slot],
                send_sem=rs_send.at[s_slot],
                recv_sem=rs_recv.at[r_slot],
                device_id=(right,),
                device_id_type=pl.DeviceIdType.MESH,
            )
            rdma.start()
            rdma.wait()
        out_ref[...] = pt[:, pl.ds(my * SQ_SHARD, SQ_SHARD), :] + rsc[1]

    return pl.pallas_call(
        body,
        out_shape=jax.ShapeDtypeStruct((B, SQ_SHARD, D), jnp.float32),
        in_specs=[pl.BlockSpec(memory_space=pltpu.VMEM)] * 5,
        out_specs=pl.BlockSpec(memory_space=pltpu.VMEM),
        scratch_shapes=[
            pltpu.VMEM((B, SQ, D), jnp.float32),
            pltpu.VMEM((B, SQ, D), jnp.float32),
            pltpu.VMEM((2, B, SQ_SHARD, D), jnp.float32),
            pltpu.VMEM((2, B, SQ_SHARD, D), jnp.float32),
            pltpu.SemaphoreType.DMA((2,)),
            pltpu.SemaphoreType.DMA((2,)),
            pltpu.SemaphoreType.DMA((2,)),
            pltpu.SemaphoreType.DMA((2,)),
        ],
        compiler_params=pltpu.CompilerParams(collective_id=0),
    )(x, Wq, Wo, k_loc, v_loc)


# device time: 163760 ns/iter; 1.6872x vs baseline; 1.6872x over previous
import jax
import jax.numpy as jnp
from jax import lax
from jax.experimental import pallas as pl
from jax.experimental.pallas import tpu as pltpu

N_DEV = 16
B = 2
SQ_SHARD = 128
SQ = N_DEV * SQ_SHARD
D = 512
H_LOC = 8
DH = 64
SKV = 128
SCALE = 0.125

F32 = jnp.float32


def kernel(x, Wq, Wo, K_ext, V_ext):
    my = lax.axis_index("i")
    k_loc = lax.dynamic_slice(
        K_ext.reshape(B, SKV, 128 * DH), (0, 0, my * (H_LOC * DH)),
        (B, SKV, H_LOC * DH))
    v_loc = lax.dynamic_slice(
        V_ext.reshape(B, SKV, 128 * DH), (0, 0, my * (H_LOC * DH)),
        (B, SKV, H_LOC * DH))

    def body(x_ref, wq_ref, wo_ref, k_ref, v_ref, out_ref,
             pt, oa, agR, agL, rsR, rsL,
             agR_s, agR_r, agL_s, agL_r, rsR_s, rsR_r, rsL_s, rsL_r):
        my = lax.axis_index("i")
        left = (my - 1) % N_DEV
        right = (my + 1) % N_DEV

        barrier_sem = pltpu.get_barrier_semaphore()
        for nbr in [left, right]:
            pl.semaphore_signal(
                barrier_sem, inc=1,
                device_id=(nbr,), device_id_type=pl.DeviceIdType.MESH)
        pl.semaphore_wait(barrier_sem, 2)

        def compute_block(xb, j):
            x2 = xb.reshape(B * SQ_SHARD, D)
            qa = jnp.dot(x2, wq_ref[...],
                         preferred_element_type=F32).reshape(B, SQ_SHARD, D)
            for h in range(H_LOC):
                c0 = h * DH
                q = qa[:, :, c0:c0 + DH]
                k = k_ref[:, :, c0:c0 + DH]
                v = v_ref[:, :, c0:c0 + DH]
                s = lax.dot_general(
                    q, k, (((2,), (2,)), ((0,), (0,))),
                    preferred_element_type=F32) * SCALE
                m = jnp.max(s, axis=-1, keepdims=True)
                p = jnp.exp(s - m)
                l = jnp.sum(p, axis=-1, keepdims=True)
                o = lax.dot_general(
                    p, v, (((2,), (1,)), ((0,), (0,))),
                    preferred_element_type=F32) / l
                oa[:, :, c0:c0 + DH] = o
            po = jnp.dot(oa[...].reshape(B * SQ_SHARD, D), wo_ref[...],
                         preferred_element_type=F32)
            pt[:, pl.ds(j * SQ_SHARD, SQ_SHARD), :] = po.reshape(
                B, SQ_SHARD, D)

        for h in range(1, 9):
            s_sl = (h - 1) % 2
            r_sl = h % 2
            rdmaR = pltpu.make_async_remote_copy(
                src_ref=(x_ref if h == 1 else agR.at[s_sl]),
                dst_ref=agR.at[r_sl],
                send_sem=agR_s.at[s_sl], recv_sem=agR_r.at[r_sl],
                device_id=(right,), device_id_type=pl.DeviceIdType.MESH)
            rdmaR.start()
            rdmaL = None
            if h <= 7:
                rdmaL = pltpu.make_async_remote_copy(
                    src_ref=(x_ref if h == 1 else agL.at[s_sl]),
                    dst_ref=agL.at[r_sl],
                    send_sem=agL_s.at[s_sl], recv_sem=agL_r.at[r_sl],
                    device_id=(left,), device_id_type=pl.DeviceIdType.MESH)
                rdmaL.start()
            if h == 1:
                compute_block(x_ref[...], my)
            else:
                compute_block(agR[(h - 1) % 2], (my - (h - 1)) % N_DEV)
                if h - 1 <= 7:
                    compute_block(agL[(h - 1) % 2], (my + (h - 1)) % N_DEV)
            rdmaR.wait()
            if rdmaL is not None:
                rdmaL.wait()
        compute_block(agR[0], (my - 8) % N_DEV)

        for st in range(1, 9):
            s_sl = (st - 1) % 2
            r_sl = st % 2
            dR = (my - 9 + st) % N_DEV
            blkR = pt[:, pl.ds(dR * SQ_SHARD, SQ_SHARD), :]
            if st == 1:
                rsR[s_sl] = blkR
            else:
                rsR[s_sl] = rsR[s_sl] + blkR
            rdmaR = pltpu.make_async_remote_copy(
                src_ref=rsR.at[s_sl], dst_ref=rsR.at[r_sl],
                send_sem=rsR_s.at[s_sl], recv_sem=rsR_r.at[r_sl],
                device_id=(left,), device_id_type=pl.DeviceIdType.MESH)
            rdmaR.start()
            rdmaL = None
            if st >= 2:
                dL = (my + 9 - st) % N_DEV
                blkL = pt[:, pl.ds(dL * SQ_SHARD, SQ_SHARD), :]
                if st == 2:
                    rsL[s_sl] = blkL
                else:
                    rsL[s_sl] = rsL[s_sl] + blkL
                rdmaL = pltpu.make_async_remote_copy(
                    src_ref=rsL.at[s_sl], dst_ref=rsL.at[r_sl],
                    send_sem=rsL_s.at[s_sl], recv_sem=rsL_r.at[r_sl],
                    device_id=(right,), device_id_type=pl.DeviceIdType.MESH)
                rdmaL.start()
            rdmaR.wait()
            if rdmaL is not None:
                rdmaL.wait()

        out_ref[...] = (pt[:, pl.ds(my * SQ_SHARD, SQ_SHARD), :]
                        + rsR[0] + rsL[0])

    blk = (B, SQ_SHARD, D)
    return pl.pallas_call(
        body,
        out_shape=jax.ShapeDtypeStruct(blk, F32),
        in_specs=[pl.BlockSpec(memory_space=pltpu.VMEM)] * 5,
        out_specs=pl.BlockSpec(memory_space=pltpu.VMEM),
        scratch_shapes=[
            pltpu.VMEM((B, SQ, D), F32),
            pltpu.VMEM((B, SQ_SHARD, D), F32),
            pltpu.VMEM((2,) + blk, F32),
            pltpu.VMEM((2,) + blk, F32),
            pltpu.VMEM((2,) + blk, F32),
            pltpu.VMEM((2,) + blk, F32),
            pltpu.SemaphoreType.DMA((2,)),
            pltpu.SemaphoreType.DMA((2,)),
            pltpu.SemaphoreType.DMA((2,)),
            pltpu.SemaphoreType.DMA((2,)),
            pltpu.SemaphoreType.DMA((2,)),
            pltpu.SemaphoreType.DMA((2,)),
            pltpu.SemaphoreType.DMA((2,)),
            pltpu.SemaphoreType.DMA((2,)),
        ],
        compiler_params=pltpu.CompilerParams(collective_id=0),
    )(x, Wq, Wo, k_loc, v_loc)


# device time: 151229 ns/iter; 1.8270x vs baseline; 1.0829x over previous
import jax
import jax.numpy as jnp
from jax import lax
from jax.experimental import pallas as pl
from jax.experimental.pallas import tpu as pltpu

N_DEV = 16
B = 2
SQ_SHARD = 128
SQ = N_DEV * SQ_SHARD
D = 512
H_LOC = 8
DH = 64
SKV = 128
SCALE = 0.125

F32 = jnp.float32

RING = [0, 1, 5, 9, 13, 14, 10, 6, 2, 3, 7, 11, 15, 12, 8, 4]
POS = [0] * N_DEV
for _r, _l in enumerate(RING):
    POS[_l] = _r


def kernel(x, Wq, Wo, K_ext, V_ext):
    my = lax.axis_index("i")
    k_loc = lax.dynamic_slice(
        K_ext.reshape(B, SKV, 128 * DH), (0, 0, my * (H_LOC * DH)),
        (B, SKV, H_LOC * DH))
    v_loc = lax.dynamic_slice(
        V_ext.reshape(B, SKV, 128 * DH), (0, 0, my * (H_LOC * DH)),
        (B, SKV, H_LOC * DH))

    ring = jnp.array(RING, dtype=jnp.int32)
    pos = jnp.array(POS, dtype=jnp.int32)[my]
    steps = jnp.arange(9, dtype=jnp.int32)
    nbrs = jnp.stack([ring[(pos - 1) % N_DEV],
                      ring[(pos + 1) % N_DEV]])
    originR = ring[(pos - steps) % N_DEV]
    originL = ring[(pos + steps) % N_DEV]
    destR = ring[(pos - 9 + steps) % N_DEV]
    destL = ring[(pos + 9 - steps) % N_DEV]

    def body(x_ref, wq_ref, wo_ref, k_ref, v_ref,
             nbr_ref, oR_ref, oL_ref, dR_ref, dL_ref, out_ref,
             pt, oa, agR, agL, rsR, rsL,
             agR_s, agR_r, agL_s, agL_r, rsR_s, rsR_r, rsL_s, rsL_r):
        my = lax.axis_index("i")
        left = nbr_ref[0]
        right = nbr_ref[1]

        barrier_sem = pltpu.get_barrier_semaphore()
        for nbr in [left, right]:
            pl.semaphore_signal(
                barrier_sem, inc=1,
                device_id=(nbr,), device_id_type=pl.DeviceIdType.MESH)
        pl.semaphore_wait(barrier_sem, 2)

        def compute_block(xb, j):
            x2 = xb.reshape(B * SQ_SHARD, D)
            qa = jnp.dot(x2, wq_ref[...],
                         preferred_element_type=F32).reshape(B, SQ_SHARD, D)
            for h in range(H_LOC):
                c0 = h * DH
                q = qa[:, :, c0:c0 + DH]
                k = k_ref[:, :, c0:c0 + DH]
                v = v_ref[:, :, c0:c0 + DH]
                s = lax.dot_general(
                    q, k, (((2,), (2,)), ((0,), (0,))),
                    preferred_element_type=F32) * SCALE
                m = jnp.max(s, axis=-1, keepdims=True)
                p = jnp.exp(s - m)
                l = jnp.sum(p, axis=-1, keepdims=True)
                o = lax.dot_general(
                    p, v, (((2,), (1,)), ((0,), (0,))),
                    preferred_element_type=F32) / l
                oa[:, :, c0:c0 + DH] = o
            po = jnp.dot(oa[...].reshape(B * SQ_SHARD, D), wo_ref[...],
                         preferred_element_type=F32)
            pt[:, pl.ds(j * SQ_SHARD, SQ_SHARD), :] = po.reshape(
                B, SQ_SHARD, D)

        for h in range(1, 9):
            s_sl = (h - 1) % 2
            r_sl = h % 2
            rdmaR = pltpu.make_async_remote_copy(
                src_ref=(x_ref if h == 1 else agR.at[s_sl]),
                dst_ref=agR.at[r_sl],
                send_sem=agR_s.at[s_sl], recv_sem=agR_r.at[r_sl],
                device_id=(right,), device_id_type=pl.DeviceIdType.MESH)
            rdmaR.start()
            rdmaL = None
            if h <= 7:
                rdmaL = pltpu.make_async_remote_copy(
                    src_ref=(x_ref if h == 1 else agL.at[s_sl]),
                    dst_ref=agL.at[r_sl],
                    send_sem=agL_s.at[s_sl], recv_sem=agL_r.at[r_sl],
                    device_id=(left,), device_id_type=pl.DeviceIdType.MESH)
                rdmaL.start()
            if h == 1:
                compute_block(x_ref[...], my)
            else:
                compute_block(agR[(h - 1) % 2], oR_ref[h - 1])
                if h - 1 <= 7:
                    compute_block(agL[(h - 1) % 2], oL_ref[h - 1])
            rdmaR.wait()
            if rdmaL is not None:
                rdmaL.wait()
        compute_block(agR[0], oR_ref[8])

        for st in range(1, 9):
            s_sl = (st - 1) % 2
            r_sl = st % 2
            dR = dR_ref[st]
            blkR = pt[:, pl.ds(dR * SQ_SHARD, SQ_SHARD), :]
            if st == 1:
                rsR[s_sl] = blkR
            else:
                rsR[s_sl] = rsR[s_sl] + blkR
            rdmaR = pltpu.make_async_remote_copy(
                src_ref=rsR.at[s_sl], dst_ref=rsR.at[r_sl],
                send_sem=rsR_s.at[s_sl], recv_sem=rsR_r.at[r_sl],
                device_id=(left,), device_id_type=pl.DeviceIdType.MESH)
            rdmaR.start()
            rdmaL = None
            if st >= 2:
                dL = dL_ref[st]
                blkL = pt[:, pl.ds(dL * SQ_SHARD, SQ_SHARD), :]
                if st == 2:
                    rsL[s_sl] = blkL
                else:
                    rsL[s_sl] = rsL[s_sl] + blkL
                rdmaL = pltpu.make_async_remote_copy(
                    src_ref=rsL.at[s_sl], dst_ref=rsL.at[r_sl],
                    send_sem=rsL_s.at[s_sl], recv_sem=rsL_r.at[r_sl],
                    device_id=(right,), device_id_type=pl.DeviceIdType.MESH)
                rdmaL.start()
            rdmaR.wait()
            if rdmaL is not None:
                rdmaL.wait()

        out_ref[...] = (pt[:, pl.ds(my * SQ_SHARD, SQ_SHARD), :]
                        + rsR[0] + rsL[0])

    blk = (B, SQ_SHARD, D)
    return pl.pallas_call(
        body,
        out_shape=jax.ShapeDtypeStruct(blk, F32),
        in_specs=[pl.BlockSpec(memory_space=pltpu.VMEM)] * 5
        + [pl.BlockSpec(memory_space=pltpu.SMEM)] * 5,
        out_specs=pl.BlockSpec(memory_space=pltpu.VMEM),
        scratch_shapes=[
            pltpu.VMEM((B, SQ, D), F32),
            pltpu.VMEM((B, SQ_SHARD, D), F32),
            pltpu.VMEM((2,) + blk, F32),
            pltpu.VMEM((2,) + blk, F32),
            pltpu.VMEM((2,) + blk, F32),
            pltpu.VMEM((2,) + blk, F32),
            pltpu.SemaphoreType.DMA((2,)),
            pltpu.SemaphoreType.DMA((2,)),
            pltpu.SemaphoreType.DMA((2,)),
            pltpu.SemaphoreType.DMA((2,)),
            pltpu.SemaphoreType.DMA((2,)),
            pltpu.SemaphoreType.DMA((2,)),
            pltpu.SemaphoreType.DMA((2,)),
            pltpu.SemaphoreType.DMA((2,)),
        ],
        compiler_params=pltpu.CompilerParams(collective_id=0),
    )(x, Wq, Wo, k_loc, v_loc, nbrs, originR, originL, destR, destL)


# device time: 138108 ns/iter; 2.0005x vs baseline; 1.0950x over previous
import jax
import jax.numpy as jnp
from jax import lax
from jax.experimental import pallas as pl
from jax.experimental.pallas import tpu as pltpu

N_DEV = 16
B = 2
SQ_SHARD = 128
SQ = N_DEV * SQ_SHARD
D = 512
H_LOC = 8
DH = 64
SKV = 128
SCALE = 0.125

F32 = jnp.float32
BF16 = jnp.bfloat16

RING = [0, 1, 5, 9, 13, 14, 10, 6, 2, 3, 7, 11, 15, 12, 8, 4]
POS = [0] * N_DEV
for _r, _l in enumerate(RING):
    POS[_l] = _r


def kernel(x, Wq, Wo, K_ext, V_ext):
    my = lax.axis_index("i")
    k_loc = lax.dynamic_slice(
        K_ext.reshape(B, SKV, 128 * DH), (0, 0, my * (H_LOC * DH)),
        (B, SKV, H_LOC * DH))
    v_loc = lax.dynamic_slice(
        V_ext.reshape(B, SKV, 128 * DH), (0, 0, my * (H_LOC * DH)),
        (B, SKV, H_LOC * DH))

    ring = jnp.array(RING, dtype=jnp.int32)
    pos = jnp.array(POS, dtype=jnp.int32)[my]
    steps = jnp.arange(9, dtype=jnp.int32)
    nbrs = jnp.stack([ring[(pos - 1) % N_DEV],
                      ring[(pos + 1) % N_DEV]])
    originR = ring[(pos - steps) % N_DEV]
    originL = ring[(pos + steps) % N_DEV]
    destR = ring[(pos - 9 + steps) % N_DEV]
    destL = ring[(pos + 9 - steps) % N_DEV]

    def body(x_ref, wq_ref, wo_ref, k_ref, v_ref,
             nbr_ref, oR_ref, oL_ref, dR_ref, dL_ref, out_ref,
             pt, oa, xb16, wq16, wo16, k16, v16, agR, agL, rsR, rsL,
             agR_s, agR_r, agL_s, agL_r, rsR_s, rsR_r, rsL_s, rsL_r):
        my = lax.axis_index("i")
        left = nbr_ref[0]
        right = nbr_ref[1]

        xb16[...] = x_ref[...].astype(BF16)
        wq16[...] = wq_ref[...].astype(BF16)
        wo16[...] = wo_ref[...].astype(BF16)
        k16[...] = k_ref[...].astype(BF16)
        v16[...] = v_ref[...].astype(BF16)

        barrier_sem = pltpu.get_barrier_semaphore()
        for nbr in [left, right]:
            pl.semaphore_signal(
                barrier_sem, inc=1,
                device_id=(nbr,), device_id_type=pl.DeviceIdType.MESH)
        pl.semaphore_wait(barrier_sem, 2)

        def compute_block(xb, j):
            x2 = xb.reshape(B * SQ_SHARD, D)
            qa = jnp.dot(x2, wq16[...],
                         preferred_element_type=F32).reshape(B, SQ_SHARD, D)
            qa16 = qa.astype(BF16)
            for h in range(H_LOC):
                c0 = h * DH
                q = qa16[:, :, c0:c0 + DH]
                k = k16[:, :, c0:c0 + DH]
                v = v16[:, :, c0:c0 + DH]
                s = lax.dot_general(
                    q, k, (((2,), (2,)), ((0,), (0,))),
                    preferred_element_type=F32) * SCALE
                m = jnp.max(s, axis=-1, keepdims=True)
                p = jnp.exp(s - m)
                l = jnp.sum(p, axis=-1, keepdims=True)
                o = lax.dot_general(
                    p.astype(BF16), v, (((2,), (1,)), ((0,), (0,))),
                    preferred_element_type=F32) / l
                oa[:, :, c0:c0 + DH] = o.astype(BF16)
            po = jnp.dot(oa[...].reshape(B * SQ_SHARD, D), wo16[...],
                         preferred_element_type=F32)
            pt[:, pl.ds(j * SQ_SHARD, SQ_SHARD), :] = po.reshape(
                B, SQ_SHARD, D)

        for h in range(1, 9):
            s_sl = (h - 1) % 2
            r_sl = h % 2
            rdmaR = pltpu.make_async_remote_copy(
                src_ref=(xb16 if h == 1 else agR.at[s_sl]),
                dst_ref=agR.at[r_sl],
                send_sem=agR_s.at[s_sl], recv_sem=agR_r.at[r_sl],
                device_id=(right,), device_id_type=pl.DeviceIdType.MESH)
            rdmaR.start()
            rdmaL = None
            if h <= 7:
                rdmaL = pltpu.make_async_remote_copy(
                    src_ref=(xb16 if h == 1 else agL.at[s_sl]),
                    dst_ref=agL.at[r_sl],
                    send_sem=agL_s.at[s_sl], recv_sem=agL_r.at[r_sl],
                    device_id=(left,), device_id_type=pl.DeviceIdType.MESH)
                rdmaL.start()
            if h == 1:
                compute_block(xb16[...], my)
            else:
                compute_block(agR[(h - 1) % 2], oR_ref[h - 1])
                if h - 1 <= 7:
                    compute_block(agL[(h - 1) % 2], oL_ref[h - 1])
            rdmaR.wait()
            if rdmaL is not None:
                rdmaL.wait()
        compute_block(agR[0], oR_ref[8])

        for st in range(1, 9):
            s_sl = (st - 1) % 2
            r_sl = st % 2
            dR = dR_ref[st]
            blkR = pt[:, pl.ds(dR * SQ_SHARD, SQ_SHARD), :]
            if st == 1:
                rsR[s_sl] = blkR
            else:
                rsR[s_sl] = rsR[s_sl] + blkR
            rdmaR = pltpu.make_async_remote_copy(
                src_ref=rsR.at[s_sl], dst_ref=rsR.at[r_sl],
                send_sem=rsR_s.at[s_sl], recv_sem=rsR_r.at[r_sl],
                device_id=(left,), device_id_type=pl.DeviceIdType.MESH)
            rdmaR.start()
            rdmaL = None
            if st >= 2:
                dL = dL_ref[st]
                blkL = pt[:, pl.ds(dL * SQ_SHARD, SQ_SHARD), :]
                if st == 2:
                    rsL[s_sl] = blkL
                else:
                    rsL[s_sl] = rsL[s_sl] + blkL
                rdmaL = pltpu.make_async_remote_copy(
                    src_ref=rsL.at[s_sl], dst_ref=rsL.at[r_sl],
                    send_sem=rsL_s.at[s_sl], recv_sem=rsL_r.at[r_sl],
                    device_id=(right,), device_id_type=pl.DeviceIdType.MESH)
                rdmaL.start()
            rdmaR.wait()
            if rdmaL is not None:
                rdmaL.wait()

        out_ref[...] = (pt[:, pl.ds(my * SQ_SHARD, SQ_SHARD), :]
                        + rsR[0] + rsL[0])

    blk = (B, SQ_SHARD, D)
    return pl.pallas_call(
        body,
        out_shape=jax.ShapeDtypeStruct(blk, F32),
        in_specs=[pl.BlockSpec(memory_space=pltpu.VMEM)] * 5
        + [pl.BlockSpec(memory_space=pltpu.SMEM)] * 5,
        out_specs=pl.BlockSpec(memory_space=pltpu.VMEM),
        scratch_shapes=[
            pltpu.VMEM((B, SQ, D), F32),
            pltpu.VMEM((B, SQ_SHARD, D), BF16),
            pltpu.VMEM(blk, BF16),
            pltpu.VMEM((D, D), BF16),
            pltpu.VMEM((D, D), BF16),
            pltpu.VMEM((B, SKV, D), BF16),
            pltpu.VMEM((B, SKV, D), BF16),
            pltpu.VMEM((2,) + blk, BF16),
            pltpu.VMEM((2,) + blk, BF16),
            pltpu.VMEM((2,) + blk, F32),
            pltpu.VMEM((2,) + blk, F32),
            pltpu.SemaphoreType.DMA((2,)),
            pltpu.SemaphoreType.DMA((2,)),
            pltpu.SemaphoreType.DMA((2,)),
            pltpu.SemaphoreType.DMA((2,)),
            pltpu.SemaphoreType.DMA((2,)),
            pltpu.SemaphoreType.DMA((2,)),
            pltpu.SemaphoreType.DMA((2,)),
            pltpu.SemaphoreType.DMA((2,)),
        ],
        compiler_params=pltpu.CompilerParams(collective_id=0),
    )(x, Wq, Wo, k_loc, v_loc, nbrs, originR, originL, destR, destL)


# device time: 115470 ns/iter; 2.3927x vs baseline; 1.1961x over previous
import jax
import jax.numpy as jnp
from jax import lax
from jax.experimental import pallas as pl
from jax.experimental.pallas import tpu as pltpu

N_DEV = 16
B = 2
SQ_SHARD = 128
SQ = N_DEV * SQ_SHARD
D = 512
H_LOC = 8
DH = 64
SKV = 128
SCALE = 0.125

F32 = jnp.float32
BF16 = jnp.bfloat16

RING = [0, 1, 5, 9, 13, 14, 10, 6, 2, 3, 7, 11, 15, 12, 8, 4]
POS = [0] * N_DEV
for _r, _l in enumerate(RING):
    POS[_l] = _r


def kernel(x, Wq, Wo, K_ext, V_ext):
    my = lax.axis_index("i")
    k_loc = lax.dynamic_slice(
        K_ext.reshape(B, SKV, 128 * DH), (0, 0, my * (H_LOC * DH)),
        (B, SKV, H_LOC * DH))
    v_loc = lax.dynamic_slice(
        V_ext.reshape(B, SKV, 128 * DH), (0, 0, my * (H_LOC * DH)),
        (B, SKV, H_LOC * DH))

    ring = jnp.array(RING, dtype=jnp.int32)
    pos = jnp.array(POS, dtype=jnp.int32)[my]
    steps = jnp.arange(9, dtype=jnp.int32)
    nbrs = jnp.stack([ring[(pos - 1) % N_DEV],
                      ring[(pos + 1) % N_DEV]])
    originR = ring[(pos - steps) % N_DEV]
    originL = ring[(pos + steps) % N_DEV]
    destR = ring[(pos - 9 + steps) % N_DEV]
    destL = ring[(pos + 9 - steps) % N_DEV]

    def body(x_ref, wq_ref, wo_ref, k_ref, v_ref,
             nbr_ref, oR_ref, oL_ref, dR_ref, dL_ref, out_ref,
             pt, oa, xb16, wq16, wo16, k16, v16, agR, agL, rsR, rsL,
             agR_s, agR_r, agL_s, agL_r, rsR_s, rsR_r, rsL_s, rsL_r):
        my = lax.axis_index("i")
        left = nbr_ref[0]
        right = nbr_ref[1]

        xb16[...] = x_ref[...].astype(BF16)
        wq16[...] = wq_ref[...].astype(BF16)
        wo16[...] = wo_ref[...].astype(BF16)
        k16[...] = k_ref[...].astype(BF16)
        v16[...] = v_ref[...].astype(BF16)

        barrier_sem = pltpu.get_barrier_semaphore()
        for nbr in [left, right]:
            pl.semaphore_signal(
                barrier_sem, inc=1,
                device_id=(nbr,), device_id_type=pl.DeviceIdType.MESH)
        pl.semaphore_wait(barrier_sem, 2)

        def compute_block(xb, j):
            x2 = xb.reshape(B * SQ_SHARD, D)
            qa = jnp.dot(x2, wq16[...],
                         preferred_element_type=F32).reshape(B, SQ_SHARD, D)
            qa16 = qa.astype(BF16)
            for h in range(H_LOC):
                c0 = h * DH
                q = qa16[:, :, c0:c0 + DH]
                k = k16[:, :, c0:c0 + DH]
                v = v16[:, :, c0:c0 + DH]
                s = lax.dot_general(
                    q, k, (((2,), (2,)), ((0,), (0,))),
                    preferred_element_type=F32) * SCALE
                m = jnp.max(s, axis=-1, keepdims=True)
                p = jnp.exp(s - m)
                l = jnp.sum(p, axis=-1, keepdims=True)
                o = lax.dot_general(
                    p.astype(BF16), v, (((2,), (1,)), ((0,), (0,))),
                    preferred_element_type=F32) / l
                oa[:, :, c0:c0 + DH] = o.astype(BF16)
            po = jnp.dot(oa[...].reshape(B * SQ_SHARD, D), wo16[...],
                         preferred_element_type=F32)
            pt[:, pl.ds(j * SQ_SHARD, SQ_SHARD), :] = po.reshape(
                B, SQ_SHARD, D)

        for h in range(1, 9):
            s_sl = (h - 1) % 2
            r_sl = h % 2
            rdmaR = pltpu.make_async_remote_copy(
                src_ref=(xb16 if h == 1 else agR.at[s_sl]),
                dst_ref=agR.at[r_sl],
                send_sem=agR_s.at[s_sl], recv_sem=agR_r.at[r_sl],
                device_id=(right,), device_id_type=pl.DeviceIdType.MESH)
            rdmaR.start()
            rdmaL = None
            if h <= 7:
                rdmaL = pltpu.make_async_remote_copy(
                    src_ref=(xb16 if h == 1 else agL.at[s_sl]),
                    dst_ref=agL.at[r_sl],
                    send_sem=agL_s.at[s_sl], recv_sem=agL_r.at[r_sl],
                    device_id=(left,), device_id_type=pl.DeviceIdType.MESH)
                rdmaL.start()
            if h == 1:
                compute_block(xb16[...], my)
            else:
                compute_block(agR[(h - 1) % 2], oR_ref[h - 1])
                if h - 1 <= 7:
                    compute_block(agL[(h - 1) % 2], oL_ref[h - 1])
            rdmaR.wait()
            if rdmaL is not None:
                rdmaL.wait()
        compute_block(agR[0], oR_ref[8])

        for st in range(1, 9):
            s_sl = (st - 1) % 2
            r_sl = st % 2
            dR = dR_ref[st]
            blkR = pt[:, pl.ds(dR * SQ_SHARD, SQ_SHARD), :]
            if st == 1:
                rsR[s_sl] = blkR.astype(BF16)
            else:
                rsR[s_sl] = (rsR[s_sl].astype(F32) + blkR).astype(BF16)
            rdmaR = pltpu.make_async_remote_copy(
                src_ref=rsR.at[s_sl], dst_ref=rsR.at[r_sl],
                send_sem=rsR_s.at[s_sl], recv_sem=rsR_r.at[r_sl],
                device_id=(left,), device_id_type=pl.DeviceIdType.MESH)
            rdmaR.start()
            rdmaL = None
            if st >= 2:
                dL = dL_ref[st]
                blkL = pt[:, pl.ds(dL * SQ_SHARD, SQ_SHARD), :]
                if st == 2:
                    rsL[s_sl] = blkL.astype(BF16)
                else:
                    rsL[s_sl] = (rsL[s_sl].astype(F32) + blkL).astype(BF16)
                rdmaL = pltpu.make_async_remote_copy(
                    src_ref=rsL.at[s_sl], dst_ref=rsL.at[r_sl],
                    send_sem=rsL_s.at[s_sl], recv_sem=rsL_r.at[r_sl],
                    device_id=(right,), device_id_type=pl.DeviceIdType.MESH)
                rdmaL.start()
            rdmaR.wait()
            if rdmaL is not None:
                rdmaL.wait()

        out_ref[...] = (pt[:, pl.ds(my * SQ_SHARD, SQ_SHARD), :]
                        + rsR[0].astype(F32) + rsL[0].astype(F32))

    blk = (B, SQ_SHARD, D)
    return pl.pallas_call(
        body,
        out_shape=jax.ShapeDtypeStruct(blk, F32),
        in_specs=[pl.BlockSpec(memory_space=pltpu.VMEM)] * 5
        + [pl.BlockSpec(memory_space=pltpu.SMEM)] * 5,
        out_specs=pl.BlockSpec(memory_space=pltpu.VMEM),
        scratch_shapes=[
            pltpu.VMEM((B, SQ, D), F32),
            pltpu.VMEM((B, SQ_SHARD, D), BF16),
            pltpu.VMEM(blk, BF16),
            pltpu.VMEM((D, D), BF16),
            pltpu.VMEM((D, D), BF16),
            pltpu.VMEM((B, SKV, D), BF16),
            pltpu.VMEM((B, SKV, D), BF16),
            pltpu.VMEM((2,) + blk, BF16),
            pltpu.VMEM((2,) + blk, BF16),
            pltpu.VMEM((2,) + blk, BF16),
            pltpu.VMEM((2,) + blk, BF16),
            pltpu.SemaphoreType.DMA((2,)),
            pltpu.SemaphoreType.DMA((2,)),
            pltpu.SemaphoreType.DMA((2,)),
            pltpu.SemaphoreType.DMA((2,)),
            pltpu.SemaphoreType.DMA((2,)),
            pltpu.SemaphoreType.DMA((2,)),
            pltpu.SemaphoreType.DMA((2,)),
            pltpu.SemaphoreType.DMA((2,)),
        ],
        compiler_params=pltpu.CompilerParams(collective_id=0),
    )(x, Wq, Wo, k_loc, v_loc, nbrs, originR, originL, destR, destL)


# device time: 115074 ns/iter; 2.4010x vs baseline; 1.0034x over previous
import jax
import jax.numpy as jnp
from jax import lax
from jax.experimental import pallas as pl
from jax.experimental.pallas import tpu as pltpu

N_DEV = 16
B = 2
SQ_SHARD = 128
SQ = N_DEV * SQ_SHARD
D = 512
H_LOC = 8
DH = 64
SKV = 128
SCALE = 0.125

F32 = jnp.float32
BF16 = jnp.bfloat16

RING = [0, 1, 5, 9, 13, 14, 10, 6, 2, 3, 7, 11, 15, 12, 8, 4]
POS = [0] * N_DEV
for _r, _l in enumerate(RING):
    POS[_l] = _r


def kernel(x, Wq, Wo, K_ext, V_ext):
    my = lax.axis_index("i")
    k_loc = lax.dynamic_slice(
        K_ext.reshape(B, SKV, 128 * DH), (0, 0, my * (H_LOC * DH)),
        (B, SKV, H_LOC * DH))
    v_loc = lax.dynamic_slice(
        V_ext.reshape(B, SKV, 128 * DH), (0, 0, my * (H_LOC * DH)),
        (B, SKV, H_LOC * DH))

    ring = jnp.array(RING, dtype=jnp.int32)
    pos = jnp.array(POS, dtype=jnp.int32)[my]
    steps = jnp.arange(9, dtype=jnp.int32)
    nbrs = jnp.stack([ring[(pos - 1) % N_DEV],
                      ring[(pos + 1) % N_DEV]])
    originR = ring[(pos - steps) % N_DEV]
    originL = ring[(pos + steps) % N_DEV]
    destR = ring[(pos - 9 + steps) % N_DEV]
    destL = ring[(pos + 9 - steps) % N_DEV]

    def body(x_ref, wq_ref, wo_ref, k_ref, v_ref,
             nbr_ref, oR_ref, oL_ref, dR_ref, dL_ref, out_ref,
             pt, oa, xb16, wq16, wo16, k16, v16, agR, agL, rsR, rsL,
             agR_s, agR_r, agL_s, agL_r, rsR_s, rsR_r, rsL_s, rsL_r):
        my = lax.axis_index("i")
        left = nbr_ref[0]
        right = nbr_ref[1]

        xb16[...] = x_ref[...].astype(BF16)

        barrier_sem = pltpu.get_barrier_semaphore()
        for nbr in [left, right]:
            pl.semaphore_signal(
                barrier_sem, inc=1,
                device_id=(nbr,), device_id_type=pl.DeviceIdType.MESH)
        pl.semaphore_wait(barrier_sem, 2)

        def compute_block(xb, j):
            x2 = xb.reshape(B * SQ_SHARD, D)
            qa = jnp.dot(x2, wq16[...],
                         preferred_element_type=F32).reshape(B, SQ_SHARD, D)
            qa16 = qa.astype(BF16)
            for h in range(H_LOC):
                c0 = h * DH
                q = qa16[:, :, c0:c0 + DH]
                k = k16[:, :, c0:c0 + DH]
                v = v16[:, :, c0:c0 + DH]
                s = lax.dot_general(
                    q, k, (((2,), (2,)), ((0,), (0,))),
                    preferred_element_type=F32) * SCALE
                m = jnp.max(s, axis=-1, keepdims=True)
                p = jnp.exp(s - m)
                l = jnp.sum(p, axis=-1, keepdims=True)
                o = lax.dot_general(
                    p.astype(BF16), v, (((2,), (1,)), ((0,), (0,))),
                    preferred_element_type=F32) / l
                oa[:, :, c0:c0 + DH] = o.astype(BF16)
            po = jnp.dot(oa[...].reshape(B * SQ_SHARD, D), wo16[...],
                         preferred_element_type=F32)
            pt[:, pl.ds(j * SQ_SHARD, SQ_SHARD), :] = po.reshape(
                B, SQ_SHARD, D)

        agR_d = {}
        agL_d = {}
        for h in range(1, 9):
            s_sl = (h - 1) % 2
            r_sl = h % 2
            if h - 2 in agR_d:
                agR_d[h - 2].wait_send()
            if h - 2 in agL_d:
                agL_d[h - 2].wait_send()
            rdmaR = pltpu.make_async_remote_copy(
                src_ref=(xb16 if h == 1 else agR.at[s_sl]),
                dst_ref=agR.at[r_sl],
                send_sem=agR_s.at[s_sl], recv_sem=agR_r.at[r_sl],
                device_id=(right,), device_id_type=pl.DeviceIdType.MESH)
            rdmaR.start()
            agR_d[h] = rdmaR
            if h <= 7:
                rdmaL = pltpu.make_async_remote_copy(
                    src_ref=(xb16 if h == 1 else agL.at[s_sl]),
                    dst_ref=agL.at[r_sl],
                    send_sem=agL_s.at[s_sl], recv_sem=agL_r.at[r_sl],
                    device_id=(left,), device_id_type=pl.DeviceIdType.MESH)
                rdmaL.start()
                agL_d[h] = rdmaL
            if h == 1:
                wq16[...] = wq_ref[...].astype(BF16)
                wo16[...] = wo_ref[...].astype(BF16)
                k16[...] = k_ref[...].astype(BF16)
                v16[...] = v_ref[...].astype(BF16)
                compute_block(xb16[...], my)
            else:
                compute_block(agR[(h - 1) % 2], oR_ref[h - 1])
                if h - 1 <= 7:
                    compute_block(agL[(h - 1) % 2], oL_ref[h - 1])
            rdmaR.wait_recv()
            if h <= 7:
                rdmaL.wait_recv()
        for h in (7, 8):
            agR_d[h].wait_send()
        agL_d[7].wait_send()
        compute_block(agR[0], oR_ref[8])

        rsR_d = {}
        rsL_d = {}
        for st in range(1, 9):
            s_sl = (st - 1) % 2
            r_sl = st % 2
            if st - 2 in rsR_d:
                rsR_d[st - 2].wait_send()
            if st - 2 in rsL_d:
                rsL_d[st - 2].wait_send()
            dR = dR_ref[st]
            blkR = pt[:, pl.ds(dR * SQ_SHARD, SQ_SHARD), :]
            if st == 1:
                rsR[s_sl] = blkR.astype(BF16)
            else:
                rsR[s_sl] = (rsR[s_sl].astype(F32) + blkR).astype(BF16)
            rdmaR = pltpu.make_async_remote_copy(
                src_ref=rsR.at[s_sl], dst_ref=rsR.at[r_sl],
                send_sem=rsR_s.at[s_sl], recv_sem=rsR_r.at[r_sl],
                device_id=(left,), device_id_type=pl.DeviceIdType.MESH)
            rdmaR.start()
            rsR_d[st] = rdmaR
            if st >= 2:
                dL = dL_ref[st]
                blkL = pt[:, pl.ds(dL * SQ_SHARD, SQ_SHARD), :]
                if st == 2:
                    rsL[s_sl] = blkL.astype(BF16)
                else:
                    rsL[s_sl] = (rsL[s_sl].astype(F32) + blkL).astype(BF16)
                rdmaL = pltpu.make_async_remote_copy(
                    src_ref=rsL.at[s_sl], dst_ref=rsL.at[r_sl],
                    send_sem=rsL_s.at[s_sl], recv_sem=rsL_r.at[r_sl],
                    device_id=(right,), device_id_type=pl.DeviceIdType.MESH)
                rdmaL.start()
                rsL_d[st] = rdmaL
            rdmaR.wait_recv()
            if st >= 2:
                rdmaL.wait_recv()
        for st in (7, 8):
            rsR_d[st].wait_send()
            rsL_d[st].wait_send()

        out_ref[...] = (pt[:, pl.ds(my * SQ_SHARD, SQ_SHARD), :]
                        + rsR[0].astype(F32) + rsL[0].astype(F32))

    blk = (B, SQ_SHARD, D)
    return pl.pallas_call(
        body,
        out_shape=jax.ShapeDtypeStruct(blk, F32),
        in_specs=[pl.BlockSpec(memory_space=pltpu.VMEM)] * 5
        + [pl.BlockSpec(memory_space=pltpu.SMEM)] * 5,
        out_specs=pl.BlockSpec(memory_space=pltpu.VMEM),
        scratch_shapes=[
            pltpu.VMEM((B, SQ, D), F32),
            pltpu.VMEM((B, SQ_SHARD, D), BF16),
            pltpu.VMEM(blk, BF16),
            pltpu.VMEM((D, D), BF16),
            pltpu.VMEM((D, D), BF16),
            pltpu.VMEM((B, SKV, D), BF16),
            pltpu.VMEM((B, SKV, D), BF16),
            pltpu.VMEM((2,) + blk, BF16),
            pltpu.VMEM((2,) + blk, BF16),
            pltpu.VMEM((2,) + blk, BF16),
            pltpu.VMEM((2,) + blk, BF16),
            pltpu.SemaphoreType.DMA((2,)),
            pltpu.SemaphoreType.DMA((2,)),
            pltpu.SemaphoreType.DMA((2,)),
            pltpu.SemaphoreType.DMA((2,)),
            pltpu.SemaphoreType.DMA((2,)),
            pltpu.SemaphoreType.DMA((2,)),
            pltpu.SemaphoreType.DMA((2,)),
            pltpu.SemaphoreType.DMA((2,)),
        ],
        compiler_params=pltpu.CompilerParams(collective_id=0),
    )(x, Wq, Wo, k_loc, v_loc, nbrs, originR, originL, destR, destL)
